# baseline (speedup 1.0000x reference)
import jax
import jax.numpy as jnp
from jax.experimental import pallas as pl
from jax.experimental.pallas import tpu as pltpu

N_NODES = 62
FEAT = 64
NPAD = 64
N_TRIL = N_NODES * (N_NODES + 1) // 2
HP = jax.lax.Precision.HIGHEST
GRID = 8
BB = 256 // GRID


def _eeg_kernel(xr_ref, p_ref, w0a_ref, w1a_ref, w0b_ref, w1b_ref,
                ba_ref, bb_ref, fcw_ref, fcb_ref, out_ref, q_ref, b_ref):
    @pl.when(pl.program_id(0) == 0)
    def _build_q():
        _build_q_body(p_ref, w0a_ref, w1a_ref, w0b_ref, w1b_ref,
                      ba_ref, bb_ref, fcw_ref, fcb_ref, q_ref, b_ref)

    out_ref[:, :] = (jnp.dot(xr_ref[:, :], q_ref[:, :], precision=HP)
                     + b_ref[:, :])


def _build_q_body(p_ref, w0a_ref, w1a_ref, w0b_ref, w1b_ref,
                  ba_ref, bb_ref, fcw_ref, fcb_ref, q_ref, b_ref):
    pv = p_ref[:, :]
    rows = []
    for r in range(N_NODES):
        off = r * (r + 1) // 2
        rows.append(jax.lax.slice(pv, (0, off), (1, off + N_NODES)))
    P62 = jnp.concatenate(rows, axis=0)
    P = jnp.concatenate(
        [jnp.concatenate([P62, jnp.zeros((N_NODES, NPAD - N_NODES), jnp.float32)], axis=1),
         jnp.zeros((NPAD - N_NODES, NPAD), jnp.float32)], axis=0)
    row_id = jax.lax.broadcasted_iota(jnp.int32, (NPAD, NPAD), 0)
    col_id = jax.lax.broadcasted_iota(jnp.int32, (NPAD, NPAD), 1)
    A = jnp.where((col_id < row_id) & (row_id < N_NODES), P, 0.0)
    A = A + A.T
    deg_c = jnp.sum(A, axis=1, keepdims=True)
    deg_r = jnp.sum(A, axis=0, keepdims=True)
    dis_c = jnp.where(deg_c > 0, 1.0 / jnp.sqrt(jnp.maximum(deg_c, 1e-12)), 0.0)
    dis_r = jnp.where(deg_r > 0, 1.0 / jnp.sqrt(jnp.maximum(deg_r, 1e-12)), 0.0)
    S = -(dis_c * A * dis_r)
    srow = jnp.sum(S, axis=0, keepdims=True)

    w0a = w0a_ref[:, :]
    w1a = w1a_ref[:, :]
    w0b = w0b_ref[:, :]
    w1b = w1b_ref[:, :]
    m0 = jnp.dot(w0a, w0b, precision=HP)
    m1 = jnp.dot(w0a, w1b, precision=HP) + jnp.dot(w1a, w0b, precision=HP)
    m2 = jnp.dot(w1a, w1b, precision=HP)
    alpha = jnp.dot(ba_ref[:, :], w0b, precision=HP) + bb_ref[:, :]
    beta = jnp.dot(ba_ref[:, :], w1b, precision=HP)

    R0 = jnp.concatenate(
        [jnp.transpose(fcw_ref[:, :]),
         jnp.zeros((NPAD - N_NODES, 3), jnp.float32)], axis=0)
    R1 = jnp.dot(S, R0, precision=HP)
    R2 = jnp.dot(S, R1, precision=HP)

    q = (m0.reshape(1, FEAT, 1) * jax.lax.slice(R0, (0, 0), (N_NODES, 3)).reshape(N_NODES, 1, 3)
         + m1.reshape(1, FEAT, 1) * jax.lax.slice(R1, (0, 0), (N_NODES, 3)).reshape(N_NODES, 1, 3)
         + m2.reshape(1, FEAT, 1) * jax.lax.slice(R2, (0, 0), (N_NODES, 3)).reshape(N_NODES, 1, 3))
    q_ref[:, :] = q.reshape(N_NODES * FEAT, 3)
    b_ref[:, :] = (jnp.dot(alpha + beta * srow, R0, precision=HP)
                   + fcb_ref[:, :])


def kernel(x, edge_index, y, batch, edge_weight_param, W0a, W1a, ba,
           W0b, W1b, bb, fcW, fcb):
    bsz = y.shape[0]
    xr = x.reshape(bsz, N_NODES * FEAT)
    p2 = edge_weight_param.reshape(1, N_TRIL)
    ba_r = ba.reshape(1, FEAT)
    bb_r = bb.reshape(1, 1)
    fcb_r = fcb.reshape(1, 3)

    full = lambda shape: pl.BlockSpec(shape, lambda i: (0, 0))
    return pl.pallas_call(
        _eeg_kernel,
        grid=(GRID,),
        in_specs=[
            pl.BlockSpec((BB, N_NODES * FEAT), lambda i: (i, 0)),
            full((1, N_TRIL)),
            full((FEAT, FEAT)), full((FEAT, FEAT)),
            full((FEAT, 1)), full((FEAT, 1)),
            full((1, FEAT)), full((1, 1)),
            full((3, N_NODES)), full((1, 3)),
        ],
        out_specs=pl.BlockSpec((BB, 3), lambda i: (i, 0)),
        scratch_shapes=[
            pltpu.VMEM((N_NODES * FEAT, 3), jnp.float32),
            pltpu.VMEM((1, 3), jnp.float32),
        ],
        out_shape=jax.ShapeDtypeStruct((bsz, 3), jnp.float32),
    )(xr, p2, W0a, W1a, W0b, W1b, ba_r, bb_r, fcW, fcb_r)


# speedup vs baseline: 1.1143x; 1.1143x over previous
import jax
import jax.numpy as jnp
from jax.experimental import pallas as pl
from jax.experimental.pallas import tpu as pltpu

N_NODES = 62
FEAT = 64
NPAD = 64
N_TRIL = N_NODES * (N_NODES + 1) // 2
HP = jax.lax.Precision.HIGHEST
GRID = 4
BB = 256 // GRID


def _eeg_kernel(xr_ref, p_ref, w0a_ref, w1a_ref, w0b_ref, w1b_ref,
                ba_ref, bb_ref, fcw_ref, fcb_ref, out_ref, q_ref, b_ref):
    @pl.when(pl.program_id(0) == 0)
    def _build_q():
        _build_q_body(p_ref, w0a_ref, w1a_ref, w0b_ref, w1b_ref,
                      ba_ref, bb_ref, fcw_ref, fcb_ref, q_ref, b_ref)

    out_ref[:, :] = (jnp.dot(xr_ref[:, :], q_ref[:, :], precision=HP)
                     + b_ref[:, :])


def _build_q_body(p_ref, w0a_ref, w1a_ref, w0b_ref, w1b_ref,
                  ba_ref, bb_ref, fcw_ref, fcb_ref, q_ref, b_ref):
    pv = p_ref[:, :]
    rows = []
    for r in range(N_NODES):
        off = r * (r + 1) // 2
        rows.append(jax.lax.slice(pv, (0, off), (1, off + N_NODES)))
    P62 = jnp.concatenate(rows, axis=0)
    P = jnp.concatenate(
        [jnp.concatenate([P62, jnp.zeros((N_NODES, NPAD - N_NODES), jnp.float32)], axis=1),
         jnp.zeros((NPAD - N_NODES, NPAD), jnp.float32)], axis=0)
    row_id = jax.lax.broadcasted_iota(jnp.int32, (NPAD, NPAD), 0)
    col_id = jax.lax.broadcasted_iota(jnp.int32, (NPAD, NPAD), 1)
    A = jnp.where((col_id < row_id) & (row_id < N_NODES), P, 0.0)
    A = A + A.T
    deg_c = jnp.sum(A, axis=1, keepdims=True)
    deg_r = jnp.sum(A, axis=0, keepdims=True)
    dis_c = jnp.where(deg_c > 0, 1.0 / jnp.sqrt(jnp.maximum(deg_c, 1e-12)), 0.0)
    dis_r = jnp.where(deg_r > 0, 1.0 / jnp.sqrt(jnp.maximum(deg_r, 1e-12)), 0.0)
    S = -(dis_c * A * dis_r)
    srow = jnp.sum(S, axis=0, keepdims=True)

    w0a = w0a_ref[:, :]
    w1a = w1a_ref[:, :]
    w0b = w0b_ref[:, :]
    w1b = w1b_ref[:, :]
    m0 = jnp.dot(w0a, w0b, precision=HP)
    m1 = jnp.dot(w0a, w1b, precision=HP) + jnp.dot(w1a, w0b, precision=HP)
    m2 = jnp.dot(w1a, w1b, precision=HP)
    alpha = jnp.dot(ba_ref[:, :], w0b, precision=HP) + bb_ref[:, :]
    beta = jnp.dot(ba_ref[:, :], w1b, precision=HP)

    R0 = jnp.concatenate(
        [jnp.transpose(fcw_ref[:, :]),
         jnp.zeros((NPAD - N_NODES, 3), jnp.float32)], axis=0)
    R1 = jnp.dot(S, R0, precision=HP)
    R2 = jnp.dot(S, R1, precision=HP)

    q = (m0.reshape(1, FEAT, 1) * jax.lax.slice(R0, (0, 0), (N_NODES, 3)).reshape(N_NODES, 1, 3)
         + m1.reshape(1, FEAT, 1) * jax.lax.slice(R1, (0, 0), (N_NODES, 3)).reshape(N_NODES, 1, 3)
         + m2.reshape(1, FEAT, 1) * jax.lax.slice(R2, (0, 0), (N_NODES, 3)).reshape(N_NODES, 1, 3))
    q_ref[:, :] = q.reshape(N_NODES * FEAT, 3)
    b_ref[:, :] = (jnp.dot(alpha + beta * srow, R0, precision=HP)
                   + fcb_ref[:, :])


def kernel(x, edge_index, y, batch, edge_weight_param, W0a, W1a, ba,
           W0b, W1b, bb, fcW, fcb):
    bsz = y.shape[0]
    xr = x.reshape(bsz, N_NODES * FEAT)
    p2 = edge_weight_param.reshape(1, N_TRIL)
    ba_r = ba.reshape(1, FEAT)
    bb_r = bb.reshape(1, 1)
    fcb_r = fcb.reshape(1, 3)

    full = lambda shape: pl.BlockSpec(shape, lambda i: (0, 0))
    return pl.pallas_call(
        _eeg_kernel,
        grid=(GRID,),
        in_specs=[
            pl.BlockSpec((BB, N_NODES * FEAT), lambda i: (i, 0)),
            full((1, N_TRIL)),
            full((FEAT, FEAT)), full((FEAT, FEAT)),
            full((FEAT, 1)), full((FEAT, 1)),
            full((1, FEAT)), full((1, 1)),
            full((3, N_NODES)), full((1, 3)),
        ],
        out_specs=pl.BlockSpec((BB, 3), lambda i: (i, 0)),
        scratch_shapes=[
            pltpu.VMEM((N_NODES * FEAT, 3), jnp.float32),
            pltpu.VMEM((1, 3), jnp.float32),
        ],
        out_shape=jax.ShapeDtypeStruct((bsz, 3), jnp.float32),
    )(xr, p2, W0a, W1a, W0b, W1b, ba_r, bb_r, fcW, fcb_r)
